# Initial kernel scaffold; baseline (speedup 1.0000x reference)
import functools

import jax
import jax.numpy as jnp
from jax import lax
from jax.experimental import pallas as pl
from jax.experimental.pallas import tpu as pltpu
from jax.experimental.pallas import tpu_sc as plsc

N = 100000
S = 1000
D = 512
DH = 1024
NC, NS, NW = 2, 16, 32
CH = 80
NCHUNK = N // CH
RB = 1000
NB = N // RB
EPS = 1e-5


def _pool_body(x_hbm, aidx_hbm, zrow_hbm, zrow2_hbm, ones_hbm,
               psums_hbm, pcnt_hbm,
               xbuf, idxbuf, onesbuf, zbuf, zbuf2, acc, cacc):
    c = lax.axis_index("c")
    s = lax.axis_index("s")
    wid = s * NC + c

    @pl.when(s < 8)
    def _():
        pltpu.sync_copy(zrow_hbm, zbuf)
        pltpu.sync_copy(zbuf, acc.at[pl.ds(s * 125, 125)])
        pltpu.sync_copy(zrow2_hbm, zbuf2)
        pltpu.sync_copy(zbuf2, cacc.at[pl.ds(s * 125, 125)])

    pltpu.sync_copy(ones_hbm, onesbuf)
    plsc.subcore_barrier()

    nmine = (NCHUNK - wid + NW - 1) // NW

    def body(i, carry):
        base = (wid + i * NW) * CH
        pltpu.sync_copy(aidx_hbm.at[pl.ds(base, CH)], idxbuf)
        pltpu.sync_copy(x_hbm.at[pl.ds(base, CH)], xbuf)
        pltpu.sync_copy(xbuf, acc.at[idxbuf], add=True)
        pltpu.sync_copy(onesbuf, cacc.at[idxbuf], add=True)
        return carry

    lax.fori_loop(0, nmine, body, 0)
    plsc.subcore_barrier()

    @pl.when(s < 8)
    def _():
        pltpu.sync_copy(acc.at[pl.ds(s * 125, 125)], zbuf)
        pltpu.sync_copy(zbuf, psums_hbm.at[pl.ds(c * S + s * 125, 125)])
        pltpu.sync_copy(cacc.at[pl.ds(s * 125, 125)], zbuf2)
        pltpu.sync_copy(zbuf2, pcnt_hbm.at[pl.ds(c * S + s * 125, 125)])


def _sc_pool(x, aidx):
    zrow = jnp.zeros((125, D), jnp.float32)
    zrow2 = jnp.zeros((125, 16), jnp.float32)
    ones = jnp.ones((CH, 16), jnp.float32)
    mesh = plsc.VectorSubcoreMesh(core_axis_name="c", subcore_axis_name="s")
    f = pl.kernel(
        _pool_body,
        out_type=(jax.ShapeDtypeStruct((NC * S, D), jnp.float32),
                  jax.ShapeDtypeStruct((NC * S, 16), jnp.float32)),
        mesh=mesh,
        scratch_types=[
            pltpu.VMEM((CH, D), jnp.float32),
            pltpu.VMEM((CH,), jnp.int32),
            pltpu.VMEM((CH, 16), jnp.float32),
            pltpu.VMEM((125, D), jnp.float32),
            pltpu.VMEM((125, 16), jnp.float32),
            pltpu.VMEM_SHARED((S, D), jnp.float32),
            pltpu.VMEM_SHARED((S, 16), jnp.float32),
        ],
    )
    return f(x, aidx, zrow, zrow2, ones)


def _gather_body(A_hbm, E_hbm, aidx_hbm, eidx_hbm, Ag_hbm, Eg_hbm,
                 buf, idxbuf, sem):
    c = lax.axis_index("c")
    s = lax.axis_index("s")
    wid = s * NC + c
    nmine = (NCHUNK - wid + NW - 1) // NW

    def body(i, carry):
        base = (wid + i * NW) * CH
        pltpu.sync_copy(aidx_hbm.at[pl.ds(base, CH)], idxbuf)
        pltpu.async_copy(A_hbm.at[idxbuf], buf, sem).wait()
        pltpu.sync_copy(buf, Ag_hbm.at[pl.ds(base, CH)])
        pltpu.sync_copy(eidx_hbm.at[pl.ds(base, CH)], idxbuf)
        pltpu.async_copy(E_hbm.at[idxbuf], buf, sem).wait()
        pltpu.sync_copy(buf, Eg_hbm.at[pl.ds(base, CH)])
        return carry

    lax.fori_loop(0, nmine, body, 0)


def _sc_gather(A, E, aidx, eidx):
    mesh = plsc.VectorSubcoreMesh(core_axis_name="c", subcore_axis_name="s")
    f = pl.kernel(
        _gather_body,
        out_type=(jax.ShapeDtypeStruct((N, DH), jnp.float32),
                  jax.ShapeDtypeStruct((N, DH), jnp.float32)),
        mesh=mesh,
        scratch_types=[
            pltpu.VMEM((CH, DH), jnp.float32),
            pltpu.VMEM((CH,), jnp.int32),
            pltpu.SemaphoreType.DMA,
        ],
    )
    return f(A, E, aidx, eidx)


def _tables_body(ps_ref, pc_ref, Wa_ref, We_ref, bA_ref, A_ref, E_ref):
    sums = ps_ref[0:S, :] + ps_ref[S:2 * S, :]
    cnt = (pc_ref[0:S, :] + pc_ref[S:2 * S, :])[:, 0:1]
    pooled = sums / jnp.maximum(cnt, 1.0)
    pa = pooled[:, 0:256]
    pe = pooled[:, 256:512]
    A_ref[...] = jnp.dot(pa, Wa_ref[...],
                         preferred_element_type=jnp.float32) + bA_ref[...]
    E_ref[...] = jnp.dot(pe, We_ref[...], preferred_element_type=jnp.float32)


def _tc_tables(psums, pcnt, Wa, We, bA):
    return pl.pallas_call(
        _tables_body,
        out_shape=(jax.ShapeDtypeStruct((S, DH), jnp.float32),
                   jax.ShapeDtypeStruct((S, DH), jnp.float32)),
    )(psums, pcnt, Wa, We, bA)


def _h0_body(x_ref, rdf_ref, bdf_ref, Wx_ref, Wdr_ref, Wdb_ref, H0_ref):
    o = jnp.dot(x_ref[...], Wx_ref[...], preferred_element_type=jnp.float32)
    dr = jnp.dot(rdf_ref[...], Wdr_ref[...], preferred_element_type=jnp.float32)
    db = jnp.dot(bdf_ref[...], Wdb_ref[...], preferred_element_type=jnp.float32)
    H0_ref[...] = o + jnp.concatenate([dr, db], axis=1)


def _tc_h0(x, rdf, bdf, Wx, Wdr, Wdb):
    return pl.pallas_call(
        _h0_body,
        grid=(NB,),
        in_specs=[
            pl.BlockSpec((RB, D), lambda i: (i, 0)),
            pl.BlockSpec((RB, 128), lambda i: (i, 0)),
            pl.BlockSpec((RB, 128), lambda i: (i, 0)),
            pl.BlockSpec((D, DH), lambda i: (0, 0)),
            pl.BlockSpec((128, D), lambda i: (0, 0)),
            pl.BlockSpec((128, D), lambda i: (0, 0)),
        ],
        out_specs=pl.BlockSpec((RB, DH), lambda i: (i, 0)),
        out_shape=jax.ShapeDtypeStruct((N, DH), jnp.float32),
    )(x, rdf, bdf, Wx, Wdr, Wdb)


def _stats1_body(H0_ref, Ag_ref, Eg_ref, st_ref):
    h = H0_ref[...] + Ag_ref[...] + Eg_ref[...]
    ssum = jnp.sum(h, axis=0, keepdims=True)
    sqsum = jnp.sum(h * h, axis=0, keepdims=True)
    blk = jnp.concatenate([ssum, sqsum], axis=0)

    @pl.when(pl.program_id(0) == 0)
    def _():
        st_ref[...] = blk

    @pl.when(pl.program_id(0) != 0)
    def _():
        st_ref[...] = st_ref[...] + blk


def _tc_stats1(H0, Ag, Eg):
    return pl.pallas_call(
        _stats1_body,
        grid=(NB,),
        in_specs=[
            pl.BlockSpec((RB, DH), lambda i: (i, 0)),
            pl.BlockSpec((RB, DH), lambda i: (i, 0)),
            pl.BlockSpec((RB, DH), lambda i: (i, 0)),
        ],
        out_specs=pl.BlockSpec((2, DH), lambda i: (0, 0)),
        out_shape=jax.ShapeDtypeStruct((2, DH), jnp.float32),
    )(H0, Ag, Eg)


def _layer2_body(H0_ref, Ag_ref, Eg_ref, st_ref, g1_ref, bt1_ref,
                 W2_ref, b2_ref, h2_ref, st2_ref):
    nf = jnp.float32(N)
    mu = st_ref[0:1, :] / nf
    var = st_ref[1:2, :] / nf - mu * mu
    rstd = lax.rsqrt(var + EPS)
    scale = g1_ref[...] * rstd
    shift = bt1_ref[...] - mu * scale
    h1 = H0_ref[...] + Ag_ref[...] + Eg_ref[...]
    x12 = jnp.maximum(h1 * scale + shift, 0.0)
    h2 = jnp.dot(x12, W2_ref[...],
                 preferred_element_type=jnp.float32) + b2_ref[...]
    h2_ref[...] = h2
    ssum = jnp.sum(h2, axis=0, keepdims=True)
    sqsum = jnp.sum(h2 * h2, axis=0, keepdims=True)
    blk = jnp.concatenate([ssum, sqsum], axis=0)

    @pl.when(pl.program_id(0) == 0)
    def _():
        st2_ref[...] = blk

    @pl.when(pl.program_id(0) != 0)
    def _():
        st2_ref[...] = st2_ref[...] + blk


def _tc_layer2(H0, Ag, Eg, st1, g1, bt1, W2, b2):
    return pl.pallas_call(
        _layer2_body,
        grid=(NB,),
        in_specs=[
            pl.BlockSpec((RB, DH), lambda i: (i, 0)),
            pl.BlockSpec((RB, DH), lambda i: (i, 0)),
            pl.BlockSpec((RB, DH), lambda i: (i, 0)),
            pl.BlockSpec((2, DH), lambda i: (0, 0)),
            pl.BlockSpec((1, DH), lambda i: (0, 0)),
            pl.BlockSpec((1, DH), lambda i: (0, 0)),
            pl.BlockSpec((DH, D), lambda i: (0, 0)),
            pl.BlockSpec((1, D), lambda i: (0, 0)),
        ],
        out_specs=(pl.BlockSpec((RB, D), lambda i: (i, 0)),
                   pl.BlockSpec((2, D), lambda i: (0, 0))),
        out_shape=(jax.ShapeDtypeStruct((N, D), jnp.float32),
                   jax.ShapeDtypeStruct((2, D), jnp.float32)),
    )(H0, Ag, Eg, st1, g1, bt1, W2, b2)


def _norm2_body(h2_ref, st2_ref, g2_ref, bt2_ref, out_ref):
    nf = jnp.float32(N)
    mu = st2_ref[0:1, :] / nf
    var = st2_ref[1:2, :] / nf - mu * mu
    rstd = lax.rsqrt(var + EPS)
    scale = g2_ref[...] * rstd
    shift = bt2_ref[...] - mu * scale
    out_ref[...] = jnp.maximum(h2_ref[...] * scale + shift, 0.0)


def _tc_norm2(h2, st2, g2, bt2):
    return pl.pallas_call(
        _norm2_body,
        grid=(NB,),
        in_specs=[
            pl.BlockSpec((RB, D), lambda i: (i, 0)),
            pl.BlockSpec((2, D), lambda i: (0, 0)),
            pl.BlockSpec((1, D), lambda i: (0, 0)),
            pl.BlockSpec((1, D), lambda i: (0, 0)),
        ],
        out_specs=pl.BlockSpec((RB, D), lambda i: (i, 0)),
        out_shape=jax.ShapeDtypeStruct((N, D), jnp.float32),
    )(h2, st2, g2, bt2)


def kernel(x, rdf_feat, bdf_feat, atom_idx, ele_idx,
           W1r, b1r, g1r, bt1r,
           W1b, b1b, g1b, bt1b,
           W2, b2, g2, bt2):
    aidx = atom_idx.astype(jnp.int32)
    eidx = ele_idx.astype(jnp.int32)

    Wx = jnp.concatenate(
        [jnp.concatenate([W1r[0:256], W1r[512:768]], axis=0),
         jnp.concatenate([W1b[0:256], W1b[512:768]], axis=0)], axis=1)
    Wdr = W1r[1024:1152]
    Wdb = W1b[1024:1152]
    Wa = jnp.concatenate([W1r[256:512], W1b[256:512]], axis=1)
    We = jnp.concatenate([W1r[768:1024], W1b[768:1024]], axis=1)
    bA = jnp.concatenate([b1r, b1b])[None, :]
    g1 = jnp.concatenate([g1r, g1b])[None, :]
    bt1 = jnp.concatenate([bt1r, bt1b])[None, :]

    psums, pcnt = _sc_pool(x, aidx)
    A, E = _tc_tables(psums, pcnt, Wa, We, bA)
    H0 = _tc_h0(x, rdf_feat, bdf_feat, Wx, Wdr, Wdb)
    Ag, Eg = _sc_gather(A, E, aidx, eidx)
    st1 = _tc_stats1(H0, Ag, Eg)
    h2, st2 = _tc_layer2(H0, Ag, Eg, st1, g1, bt1, W2, b2[None, :])
    return _tc_norm2(h2, st2, g2[None, :], bt2[None, :])


# speedup vs baseline: 1.0754x; 1.0754x over previous
import functools

import jax
import jax.numpy as jnp
from jax import lax
from jax.experimental import pallas as pl
from jax.experimental.pallas import tpu as pltpu
from jax.experimental.pallas import tpu_sc as plsc

N = 100000
S = 1000
SP = 1024
D = 512
DH = 1024
NC, NS, NW = 2, 16, 32
CH = 80
NCHUNK = N // CH
RB = 1000
NB = N // RB
EPS = 1e-5


def _gather_body(A_hbm, E_hbm, aidx_hbm, eidx_hbm, Ag_hbm, Eg_hbm,
                 buf, idxbuf, sem):
    c = lax.axis_index("c")
    s = lax.axis_index("s")
    wid = s * NC + c
    nmine = (NCHUNK - wid + NW - 1) // NW

    def body(i, carry):
        base = (wid + i * NW) * CH
        pltpu.sync_copy(aidx_hbm.at[pl.ds(base, CH)], idxbuf)
        pltpu.async_copy(A_hbm.at[idxbuf], buf, sem).wait()
        pltpu.sync_copy(buf, Ag_hbm.at[pl.ds(base, CH)])
        pltpu.sync_copy(eidx_hbm.at[pl.ds(base, CH)], idxbuf)
        pltpu.async_copy(E_hbm.at[idxbuf], buf, sem).wait()
        pltpu.sync_copy(buf, Eg_hbm.at[pl.ds(base, CH)])
        return carry

    lax.fori_loop(0, nmine, body, 0)


def _sc_gather(A, E, aidx, eidx):
    mesh = plsc.VectorSubcoreMesh(core_axis_name="c", subcore_axis_name="s")
    f = pl.kernel(
        _gather_body,
        out_type=(jax.ShapeDtypeStruct((N, DH), jnp.float32),
                  jax.ShapeDtypeStruct((N, DH), jnp.float32)),
        mesh=mesh,
        scratch_types=[
            pltpu.VMEM((CH, DH), jnp.float32),
            pltpu.VMEM((CH,), jnp.int32),
            pltpu.SemaphoreType.DMA,
        ],
    )
    return f(A, E, aidx, eidx)


def _tables_body(ps_ref, pc_ref, Wa_ref, We_ref, bA_ref, A_ref, E_ref):
    sums = ps_ref[0:S, :]
    cnt = pc_ref[0:S, 0:1].astype(jnp.float32)
    pooled = sums / jnp.maximum(cnt, 1.0)
    pa = pooled[:, 0:256]
    pe = pooled[:, 256:512]
    A_ref[...] = jnp.dot(pa, Wa_ref[...],
                         preferred_element_type=jnp.float32) + bA_ref[...]
    E_ref[...] = jnp.dot(pe, We_ref[...], preferred_element_type=jnp.float32)


def _tc_tables(psums, pcnt, Wa, We, bA):
    return pl.pallas_call(
        _tables_body,
        out_shape=(jax.ShapeDtypeStruct((S, DH), jnp.float32),
                   jax.ShapeDtypeStruct((S, DH), jnp.float32)),
    )(psums, pcnt, Wa, We, bA)


def _h0pool_body(x_ref, rdf_ref, bdf_ref, idx_ref, Wx_ref, Wdr_ref, Wdb_ref,
                 H0_ref, ps_ref, pc_ref):
    xb = x_ref[...]
    o = jnp.dot(xb, Wx_ref[...], preferred_element_type=jnp.float32)
    dr = jnp.dot(rdf_ref[...], Wdr_ref[...], preferred_element_type=jnp.float32)
    db = jnp.dot(bdf_ref[...], Wdb_ref[...], preferred_element_type=jnp.float32)
    H0_ref[...] = o + jnp.concatenate([dr, db], axis=1)

    ids = idx_ref[0]
    ohT = (lax.broadcasted_iota(jnp.int32, (SP, RB), 0)
           == jnp.broadcast_to(ids, (SP, RB))).astype(jnp.bfloat16)
    ps = jnp.dot(ohT, xb.astype(jnp.bfloat16),
                 preferred_element_type=jnp.float32)
    pc = jnp.dot(ohT, jnp.ones((RB, 8), jnp.bfloat16),
                 preferred_element_type=jnp.float32)

    @pl.when(pl.program_id(0) == 0)
    def _():
        ps_ref[...] = ps
        pc_ref[...] = pc

    @pl.when(pl.program_id(0) != 0)
    def _():
        ps_ref[...] = ps_ref[...] + ps
        pc_ref[...] = pc_ref[...] + pc


def _tc_h0pool(x, rdf, bdf, aidx3, Wx, Wdr, Wdb):
    return pl.pallas_call(
        _h0pool_body,
        grid=(NB,),
        in_specs=[
            pl.BlockSpec((RB, D), lambda i: (i, 0)),
            pl.BlockSpec((RB, 128), lambda i: (i, 0)),
            pl.BlockSpec((RB, 128), lambda i: (i, 0)),
            pl.BlockSpec((1, 1, RB), lambda i: (i, 0, 0)),
            pl.BlockSpec((D, DH), lambda i: (0, 0)),
            pl.BlockSpec((128, D), lambda i: (0, 0)),
            pl.BlockSpec((128, D), lambda i: (0, 0)),
        ],
        out_specs=(pl.BlockSpec((RB, DH), lambda i: (i, 0)),
                   pl.BlockSpec((SP, D), lambda i: (0, 0)),
                   pl.BlockSpec((SP, 8), lambda i: (0, 0))),
        out_shape=(jax.ShapeDtypeStruct((N, DH), jnp.float32),
                   jax.ShapeDtypeStruct((SP, D), jnp.float32),
                   jax.ShapeDtypeStruct((SP, 8), jnp.float32)),
    )(x, rdf, bdf, aidx3, Wx, Wdr, Wdb)


def _stats1_body(H0_ref, Ag_ref, Eg_ref, st_ref):
    h = H0_ref[...] + Ag_ref[...] + Eg_ref[...]
    ssum = jnp.sum(h, axis=0, keepdims=True)
    sqsum = jnp.sum(h * h, axis=0, keepdims=True)
    blk = jnp.concatenate([ssum, sqsum], axis=0)

    @pl.when(pl.program_id(0) == 0)
    def _():
        st_ref[...] = blk

    @pl.when(pl.program_id(0) != 0)
    def _():
        st_ref[...] = st_ref[...] + blk


def _tc_stats1(H0, Ag, Eg):
    return pl.pallas_call(
        _stats1_body,
        grid=(NB,),
        in_specs=[
            pl.BlockSpec((RB, DH), lambda i: (i, 0)),
            pl.BlockSpec((RB, DH), lambda i: (i, 0)),
            pl.BlockSpec((RB, DH), lambda i: (i, 0)),
        ],
        out_specs=pl.BlockSpec((2, DH), lambda i: (0, 0)),
        out_shape=jax.ShapeDtypeStruct((2, DH), jnp.float32),
    )(H0, Ag, Eg)


def _layer2_body(H0_ref, Ag_ref, Eg_ref, st_ref, g1_ref, bt1_ref,
                 W2_ref, b2_ref, h2_ref, st2_ref):
    nf = jnp.float32(N)
    mu = st_ref[0:1, :] / nf
    var = st_ref[1:2, :] / nf - mu * mu
    rstd = lax.rsqrt(var + EPS)
    scale = g1_ref[...] * rstd
    shift = bt1_ref[...] - mu * scale
    h1 = H0_ref[...] + Ag_ref[...] + Eg_ref[...]
    x12 = jnp.maximum(h1 * scale + shift, 0.0)
    h2 = jnp.dot(x12, W2_ref[...],
                 preferred_element_type=jnp.float32) + b2_ref[...]
    h2_ref[...] = h2
    ssum = jnp.sum(h2, axis=0, keepdims=True)
    sqsum = jnp.sum(h2 * h2, axis=0, keepdims=True)
    blk = jnp.concatenate([ssum, sqsum], axis=0)

    @pl.when(pl.program_id(0) == 0)
    def _():
        st2_ref[...] = blk

    @pl.when(pl.program_id(0) != 0)
    def _():
        st2_ref[...] = st2_ref[...] + blk


def _tc_layer2(H0, Ag, Eg, st1, g1, bt1, W2, b2):
    return pl.pallas_call(
        _layer2_body,
        grid=(NB,),
        in_specs=[
            pl.BlockSpec((RB, DH), lambda i: (i, 0)),
            pl.BlockSpec((RB, DH), lambda i: (i, 0)),
            pl.BlockSpec((RB, DH), lambda i: (i, 0)),
            pl.BlockSpec((2, DH), lambda i: (0, 0)),
            pl.BlockSpec((1, DH), lambda i: (0, 0)),
            pl.BlockSpec((1, DH), lambda i: (0, 0)),
            pl.BlockSpec((DH, D), lambda i: (0, 0)),
            pl.BlockSpec((1, D), lambda i: (0, 0)),
        ],
        out_specs=(pl.BlockSpec((RB, D), lambda i: (i, 0)),
                   pl.BlockSpec((2, D), lambda i: (0, 0))),
        out_shape=(jax.ShapeDtypeStruct((N, D), jnp.float32),
                   jax.ShapeDtypeStruct((2, D), jnp.float32)),
    )(H0, Ag, Eg, st1, g1, bt1, W2, b2)


def _norm2_body(h2_ref, st2_ref, g2_ref, bt2_ref, out_ref):
    nf = jnp.float32(N)
    mu = st2_ref[0:1, :] / nf
    var = st2_ref[1:2, :] / nf - mu * mu
    rstd = lax.rsqrt(var + EPS)
    scale = g2_ref[...] * rstd
    shift = bt2_ref[...] - mu * scale
    out_ref[...] = jnp.maximum(h2_ref[...] * scale + shift, 0.0)


def _tc_norm2(h2, st2, g2, bt2):
    return pl.pallas_call(
        _norm2_body,
        grid=(NB,),
        in_specs=[
            pl.BlockSpec((RB, D), lambda i: (i, 0)),
            pl.BlockSpec((2, D), lambda i: (0, 0)),
            pl.BlockSpec((1, D), lambda i: (0, 0)),
            pl.BlockSpec((1, D), lambda i: (0, 0)),
        ],
        out_specs=pl.BlockSpec((RB, D), lambda i: (i, 0)),
        out_shape=jax.ShapeDtypeStruct((N, D), jnp.float32),
    )(h2, st2, g2, bt2)


def kernel(x, rdf_feat, bdf_feat, atom_idx, ele_idx,
           W1r, b1r, g1r, bt1r,
           W1b, b1b, g1b, bt1b,
           W2, b2, g2, bt2):
    aidx = atom_idx.astype(jnp.int32)
    eidx = ele_idx.astype(jnp.int32)

    Wx = jnp.concatenate(
        [jnp.concatenate([W1r[0:256], W1r[512:768]], axis=0),
         jnp.concatenate([W1b[0:256], W1b[512:768]], axis=0)], axis=1)
    Wdr = W1r[1024:1152]
    Wdb = W1b[1024:1152]
    Wa = jnp.concatenate([W1r[256:512], W1b[256:512]], axis=1)
    We = jnp.concatenate([W1r[768:1024], W1b[768:1024]], axis=1)
    bA = jnp.concatenate([b1r, b1b])[None, :]
    g1 = jnp.concatenate([g1r, g1b])[None, :]
    bt1 = jnp.concatenate([bt1r, bt1b])[None, :]

    aidx3 = aidx.reshape(NB, 1, RB)
    H0, psums, pcnt = _tc_h0pool(x, rdf_feat, bdf_feat, aidx3, Wx, Wdr, Wdb)
    A, E = _tc_tables(psums, pcnt, Wa, We, bA)
    Ag, Eg = _sc_gather(A, E, aidx, eidx)
    st1 = _tc_stats1(H0, Ag, Eg)
    h2, st2 = _tc_layer2(H0, Ag, Eg, st1, g1, bt1, W2, b2[None, :])
    return _tc_norm2(h2, st2, g2[None, :], bt2[None, :])
